# baseline (speedup 1.0000x reference)
import functools

import jax
import jax.numpy as jnp
from jax import lax
from jax.experimental import pallas as pl
from jax.experimental.pallas import tpu as pltpu
from jax.experimental.pallas import tpu_sc as plsc

DIM = 32
W = 128
NW = 32
LANES = 16


def _sc_gather(weight, xt_flat, n, b):
    mesh = plsc.VectorSubcoreMesh(core_axis_name="core",
                                  subcore_axis_name="subcore")
    n_win = n // W
    per_w = n_win // NW
    wpt = b // W

    @functools.partial(
        pl.kernel,
        out_type=jax.ShapeDtypeStruct((n // b * DIM, b), jnp.float32),
        mesh=mesh,
        compiler_params=pltpu.CompilerParams(
            use_tc_tiling_on_sc=False, needs_layout_passes=False),
        scratch_types=[
            pltpu.VMEM((per_w * W,), jnp.int32),
            pltpu.VMEM((W, DIM), jnp.float32),
            pltpu.VMEM((W, DIM), jnp.float32),
            pltpu.VMEM((DIM, W), jnp.float32),
            pltpu.VMEM((DIM, W), jnp.float32),
            pltpu.SemaphoreType.DMA,
            pltpu.SemaphoreType.DMA,
            pltpu.SemaphoreType.DMA,
            pltpu.SemaphoreType.DMA,
        ],
    )
    def gather_kernel(w_hbm, x_hbm, o_hbm, idx_all, g0, g1, t0, t1,
                      gs0, gs1, os0, os1):
        wid = lax.axis_index("subcore") * 2 + lax.axis_index("core")
        base_win = wid * per_w
        pltpu.sync_copy(x_hbm.at[pl.ds(base_win * W, per_w * W)], idx_all)

        gbufs = (g0, g1)
        tbufs = (t0, t1)
        gsems = (gs0, gs1)
        osems = (os0, os1)
        rows = [lax.iota(jnp.int32, LANES) + k * LANES
                for k in range(W // LANES)]

        def out_slice(k):
            win = base_win + k
            c = lax.rem(win, wpt)
            return o_hbm.at[pl.ds((win - c) * DIM // wpt, DIM),
                            pl.ds(c * W, W)]

        def start_gather(k, p):
            pltpu.async_copy(
                w_hbm.at[idx_all.at[pl.ds(k * W, W)]], gbufs[p], gsems[p])

        def wait_gather(p):
            pltpu.make_async_copy(w_hbm.at[idx_all.at[pl.ds(0, W)]],
                                  gbufs[p], gsems[p]).wait()

        def start_out(k, p):
            pltpu.async_copy(tbufs[p], out_slice(k), osems[p])

        def wait_out(k, p):
            pltpu.make_async_copy(tbufs[p], out_slice(k), osems[p]).wait()

        def transpose(p):
            gb = gbufs[p]
            tb = tbufs[p]

            @pl.loop(0, DIM)
            def _(d):
                col = jnp.full((LANES,), d, dtype=jnp.int32)
                for k in range(W // LANES):
                    tb[d, pl.ds(k * LANES, LANES)] = plsc.load_gather(
                        gb, [rows[k], col])

        start_gather(0, 0)

        @pl.loop(0, per_w, step=2)
        def _(k):
            for h in range(2):
                kk = k + h

                @pl.when(kk + 1 < per_w)
                def _():
                    start_gather(kk + 1, (h + 1) % 2)

                wait_gather(h)

                @pl.when(kk >= 2)
                def _():
                    wait_out(kk - 2, h)

                transpose(h)
                start_out(kk, h)

        wait_out(per_w - 2, 0)
        wait_out(per_w - 1, 1)

    return gather_kernel(weight, xt_flat)


def kernel(x, weight):
    b, t = x.shape
    n = x.size
    xt_flat = x.T.reshape((n,)).astype(jnp.int32)
    out2 = _sc_gather(weight, xt_flat, n, b)
    return out2.reshape(t, DIM, b).transpose((2, 0, 1))


# speedup vs baseline: 1.0007x; 1.0007x over previous
import functools

import jax
import jax.numpy as jnp
from jax import lax
from jax.experimental import pallas as pl
from jax.experimental.pallas import tpu as pltpu
from jax.experimental.pallas import tpu_sc as plsc

DIM = 32
W = 512
TPW = 4
NW = 32
LANES = 16


def _sc_gather(weight, xp_flat, n, b, t):
    mesh = plsc.VectorSubcoreMesh(core_axis_name="core",
                                  subcore_axis_name="subcore")
    n_win = n // W
    per_w = n_win // NW
    bt_n = b // 128
    wpt = bt_n * 2

    @functools.partial(
        pl.kernel,
        out_type=jax.ShapeDtypeStruct((t, DIM, b), jnp.float32),
        mesh=mesh,
        compiler_params=pltpu.CompilerParams(
            use_tc_tiling_on_sc=False, needs_layout_passes=False),
        scratch_types=[
            pltpu.VMEM((per_w * W,), jnp.int32),
            pltpu.VMEM((W, DIM), jnp.float32),
            pltpu.VMEM((W, DIM), jnp.float32),
            pltpu.VMEM((TPW, DIM, 128), jnp.float32),
            pltpu.VMEM((TPW, DIM, 128), jnp.float32),
            pltpu.SemaphoreType.DMA,
            pltpu.SemaphoreType.DMA,
            pltpu.SemaphoreType.DMA,
            pltpu.SemaphoreType.DMA,
        ],
    )
    def gather_kernel(w_hbm, x_hbm, o_hbm, idx_all, g0, g1, t0b, t1b,
                      gs0, gs1, os0, os1):
        wid = lax.axis_index("subcore") * 2 + lax.axis_index("core")
        base_win = wid * per_w
        pltpu.sync_copy(x_hbm.at[pl.ds(base_win * W, per_w * W)], idx_all)

        gbufs = (g0, g1)
        tbufs = (t0b, t1b)
        gsems = (gs0, gs1)
        osems = (os0, os1)
        rows = [lax.iota(jnp.int32, LANES) + k * LANES
                for k in range(W // LANES)]

        def out_slice(k):
            win = base_win + k
            r = lax.rem(win, wpt)
            tt = win // wpt
            t0 = tt * 8 + lax.rem(win, 2) * TPW
            b0 = (r // 2) * 128
            return o_hbm.at[pl.ds(t0, TPW), :, pl.ds(b0, 128)]

        def start_gather(k, p):
            pltpu.async_copy(
                w_hbm.at[idx_all.at[pl.ds(k * W, W)]], gbufs[p], gsems[p])

        def wait_gather(p):
            pltpu.make_async_copy(w_hbm.at[idx_all.at[pl.ds(0, W)]],
                                  gbufs[p], gsems[p]).wait()

        def start_out(k, p):
            pltpu.async_copy(tbufs[p], out_slice(k), osems[p])

        def wait_out(k, p):
            pltpu.make_async_copy(tbufs[p], out_slice(k), osems[p]).wait()

        def transpose(p):
            gb = gbufs[p]
            tb = tbufs[p]

            @pl.loop(0, DIM)
            def _(d):
                col = jnp.full((LANES,), d, dtype=jnp.int32)
                for s in range(TPW):
                    for k in range(128 // LANES):
                        tb[s, d, pl.ds(k * LANES, LANES)] = plsc.load_gather(
                            gb, [rows[s * 8 + k], col])

        start_gather(0, 0)

        @pl.loop(0, per_w, step=2)
        def _(k):
            for h in range(2):
                kk = k + h

                @pl.when(kk + 1 < per_w)
                def _():
                    start_gather(kk + 1, (h + 1) % 2)

                wait_gather(h)

                @pl.when(kk >= 2)
                def _():
                    wait_out(kk - 2, h)

                transpose(h)
                start_out(kk, h)

        wait_out(per_w - 2, 0)
        wait_out(per_w - 1, 1)

    return gather_kernel(weight, xp_flat)


def kernel(x, weight):
    b, t = x.shape
    n = x.size
    xp = x.reshape(b // 128, 128, t // 8, 8).transpose((2, 0, 3, 1))
    xp_flat = xp.reshape((n,)).astype(jnp.int32)
    out3 = _sc_gather(weight, xp_flat, n, b, t)
    return out3.transpose((2, 0, 1))


# speedup vs baseline: 1.0998x; 1.0991x over previous
import functools

import jax
import jax.numpy as jnp
from jax import lax
from jax.experimental import pallas as pl
from jax.experimental.pallas import tpu as pltpu
from jax.experimental.pallas import tpu_sc as plsc

DIM = 32
W = 512
TPW = 4
NW = 32
LANES = 16


def _sc_gather(weight, xp_flat, n, b, t):
    mesh = plsc.VectorSubcoreMesh(core_axis_name="core",
                                  subcore_axis_name="subcore")
    n_win = n // W
    per_w = n_win // NW
    bt_n = b // 128
    wpt = bt_n * 2

    @functools.partial(
        pl.kernel,
        out_type=jax.ShapeDtypeStruct((t, DIM, b), jnp.float32),
        mesh=mesh,
        compiler_params=pltpu.CompilerParams(
            use_tc_tiling_on_sc=False, needs_layout_passes=False,
            disable_bounds_checks=True),
        scratch_types=[
            pltpu.VMEM((per_w * W,), jnp.int32),
            pltpu.VMEM((W, DIM), jnp.float32),
            pltpu.VMEM((W, DIM), jnp.float32),
            pltpu.VMEM((TPW, DIM, 128), jnp.float32),
            pltpu.VMEM((TPW, DIM, 128), jnp.float32),
            pltpu.SemaphoreType.DMA,
            pltpu.SemaphoreType.DMA,
            pltpu.SemaphoreType.DMA,
            pltpu.SemaphoreType.DMA,
        ],
    )
    def gather_kernel(w_hbm, x_hbm, o_hbm, idx_all, g0, g1, t0b, t1b,
                      gs0, gs1, os0, os1):
        wid = lax.axis_index("subcore") * 2 + lax.axis_index("core")
        base_win = wid * per_w
        pltpu.sync_copy(x_hbm.at[pl.ds(base_win * W, per_w * W)], idx_all)

        gbufs = (g0, g1)
        tbufs = (t0b, t1b)
        gsems = (gs0, gs1)
        osems = (os0, os1)
        rows = [lax.iota(jnp.int32, LANES) + k * LANES
                for k in range(W // LANES)]

        def out_slice(k):
            win = base_win + k
            r = lax.rem(win, wpt)
            tt = win // wpt
            t0 = tt * 8 + lax.rem(win, 2) * TPW
            b0 = (r // 2) * 128
            return o_hbm.at[pl.ds(t0, TPW), :, pl.ds(b0, 128)]

        def start_gather(k, p):
            pltpu.async_copy(
                w_hbm.at[idx_all.at[pl.ds(k * W, W)]], gbufs[p], gsems[p])

        def wait_gather(p):
            pltpu.make_async_copy(w_hbm.at[idx_all.at[pl.ds(0, W)]],
                                  gbufs[p], gsems[p]).wait()

        def start_out(k, p):
            pltpu.async_copy(tbufs[p], out_slice(k), osems[p])

        def wait_out(k, p):
            pltpu.make_async_copy(tbufs[p], out_slice(k), osems[p]).wait()

        def transpose(p):
            gb = gbufs[p]
            tb = tbufs[p]

            @pl.loop(0, DIM)
            def _(d):
                col = jnp.full((LANES,), d, dtype=jnp.int32)
                vals = [plsc.load_gather(gb, [rows[s * 8 + k], col])
                        for s in range(TPW) for k in range(128 // LANES)]
                for s in range(TPW):
                    for k in range(128 // LANES):
                        tb[s, d, pl.ds(k * LANES, LANES)] = (
                            vals[s * 8 + k])

        start_gather(0, 0)

        @pl.loop(0, per_w, step=2)
        def _(k):
            for h in range(2):
                kk = k + h

                @pl.when(kk + 1 < per_w)
                def _():
                    start_gather(kk + 1, (h + 1) % 2)

                wait_gather(h)

                @pl.when(kk >= 2)
                def _():
                    wait_out(kk - 2, h)

                transpose(h)
                start_out(kk, h)

        wait_out(per_w - 2, 0)
        wait_out(per_w - 1, 1)

    return gather_kernel(weight, xp_flat)


def kernel(x, weight):
    b, t = x.shape
    n = x.size
    xp = x.reshape(b // 128, 128, t // 8, 8).transpose((2, 0, 3, 1))
    xp_flat = xp.reshape((n,)).astype(jnp.int32)
    out3 = _sc_gather(weight, xp_flat, n, b, t)
    return out3.transpose((2, 0, 1))


# speedup vs baseline: 1.2107x; 1.1009x over previous
import functools

import jax
import jax.numpy as jnp
from jax import lax
from jax.experimental import pallas as pl
from jax.experimental.pallas import tpu as pltpu
from jax.experimental.pallas import tpu_sc as plsc

DIM = 32
W = 256
TPW = 2
NW = 32
LANES = 16
GROUPS = W // LANES


def _sc_gather(w128, xp_flat, n, b, t):
    mesh = plsc.VectorSubcoreMesh(core_axis_name="core",
                                  subcore_axis_name="subcore")
    n_win = n // W
    per_w = n_win // NW

    @functools.partial(
        pl.kernel,
        out_type=jax.ShapeDtypeStruct((t, DIM, b), jnp.float32),
        mesh=mesh,
        compiler_params=pltpu.CompilerParams(
            needs_layout_passes=False, disable_bounds_checks=True),
        scratch_types=[
            pltpu.VMEM((per_w * W,), jnp.int32),
            pltpu.VMEM((W,), jnp.int32),
            pltpu.VMEM((W,), jnp.int32),
            pltpu.VMEM((W, 128), jnp.float32),
            pltpu.VMEM((W, 128), jnp.float32),
            pltpu.VMEM((TPW, DIM, 128), jnp.float32),
            pltpu.VMEM((TPW, DIM, 128), jnp.float32),
            pltpu.SemaphoreType.DMA,
            pltpu.SemaphoreType.DMA,
            pltpu.SemaphoreType.DMA,
            pltpu.SemaphoreType.DMA,
        ],
    )
    def gather_kernel(w_hbm, x_hbm, o_hbm, idx_all, q0, q1, g0, g1,
                      t0b, t1b, gs0, gs1, os0, os1):
        wid = lax.axis_index("subcore") * 2 + lax.axis_index("core")
        base_win = wid * per_w
        pltpu.sync_copy(x_hbm.at[pl.ds(base_win * W, per_w * W)], idx_all)

        qbufs = (q0, q1)
        gbufs = (g0, g1)
        tbufs = (t0b, t1b)
        gsems = (gs0, gs1)
        osems = (os0, os1)
        rows = [lax.iota(jnp.int32, LANES) + g * LANES
                for g in range(GROUPS)]

        def out_slice(k):
            win = base_win + k
            tt = win // 128
            bt = lax.rem(win, 128) // 4
            t0 = tt * 8 + lax.rem(win, 4) * TPW
            return o_hbm.at[pl.ds(t0, TPW), :, pl.ds(bt * 128, 128)]

        def prep_gather(k, p):
            @pl.loop(0, GROUPS)
            def _(j):
                qbufs[p][pl.ds(j * LANES, LANES)] = (
                    idx_all[pl.ds(k * W + j * LANES, LANES)] >> 2)

            pltpu.async_copy(w_hbm.at[qbufs[p]], gbufs[p], gsems[p])

        def wait_gather(p):
            pltpu.make_async_copy(w_hbm.at[qbufs[p]], gbufs[p],
                                  gsems[p]).wait()

        def start_out(k, p):
            pltpu.async_copy(tbufs[p], out_slice(k), osems[p])

        def wait_out(k, p):
            pltpu.make_async_copy(tbufs[p], out_slice(k), osems[p]).wait()

        def extract(k, p):
            gb = gbufs[p]
            tb = tbufs[p]
            cols = [(idx_all[pl.ds(k * W + g * LANES, LANES)] & 3) * DIM
                    for g in range(GROUPS)]

            @pl.loop(0, DIM)
            def _(d):
                vals = [plsc.load_gather(gb, [rows[g], cols[g] + d])
                        for g in range(GROUPS)]
                for g in range(GROUPS):
                    s, j = divmod(g, 128 // LANES)
                    tb[s, d, pl.ds(j * LANES, LANES)] = vals[g]

        prep_gather(0, 0)

        @pl.loop(0, per_w, step=2)
        def _(k):
            for h in range(2):
                kk = k + h

                @pl.when(kk + 1 < per_w)
                def _():
                    prep_gather(kk + 1, (h + 1) % 2)

                wait_gather(h)

                @pl.when(kk >= 2)
                def _():
                    wait_out(kk - 2, h)

                extract(kk, h)
                start_out(kk, h)

        wait_out(per_w - 2, 0)
        wait_out(per_w - 1, 1)

    return gather_kernel(w128, xp_flat)


def kernel(x, weight):
    b, t = x.shape
    n = x.size
    xp = x.reshape(b // 128, 128, t // 8, 8).transpose((2, 0, 3, 1))
    xp_flat = xp.reshape((n,)).astype(jnp.int32)
    w128 = weight.reshape(weight.shape[0] // 4, DIM * 4)
    out3 = _sc_gather(w128, xp_flat, n, b, t)
    return out3.transpose((2, 0, 1))
